# baseline (device time: 15131 ns/iter reference)
import jax
import jax.numpy as jnp
from jax import lax
from jax.experimental import pallas as pl
from jax.experimental.pallas import tpu as pltpu

N_DEV = 4
HALO = 3


def _halo_exchange(x):
    b, s_per, c = x.shape

    def body(x_hbm, halo_out, send_sem, recv_sem):
        my_i = lax.axis_index("i")
        left = (my_i - 1) % N_DEV
        right = (my_i + 1) % N_DEV

        barrier_sem = pltpu.get_barrier_semaphore()
        for nbr in [left, right]:
            pl.semaphore_signal(
                barrier_sem, inc=1,
                device_id=(nbr,), device_id_type=pl.DeviceIdType.MESH,
            )
        pl.semaphore_wait(barrier_sem, 2)

        @pl.when(my_i < N_DEV - 1)
        def _send():
            rdma = pltpu.make_async_remote_copy(
                src_ref=x_hbm.at[:, pl.ds(s_per - HALO, HALO), :],
                dst_ref=halo_out,
                send_sem=send_sem,
                recv_sem=recv_sem,
                device_id=(right,),
                device_id_type=pl.DeviceIdType.MESH,
            )
            rdma.start()
            rdma.wait_send()

        @pl.when(my_i > 0)
        def _recv():
            recv = pltpu.make_async_remote_copy(
                src_ref=x_hbm.at[:, pl.ds(s_per - HALO, HALO), :],
                dst_ref=halo_out,
                send_sem=send_sem,
                recv_sem=recv_sem,
                device_id=(left,),
                device_id_type=pl.DeviceIdType.MESH,
            )
            recv.wait_recv()

        @pl.when(my_i == 0)
        def _zero():
            halo_out[...] = jnp.zeros((b, HALO, c), jnp.float32)

    return pl.pallas_call(
        body,
        out_shape=jax.ShapeDtypeStruct((b, HALO, c), jnp.float32),
        in_specs=[pl.BlockSpec(memory_space=pl.ANY)],
        out_specs=pl.BlockSpec(memory_space=pltpu.VMEM),
        scratch_shapes=[
            pltpu.SemaphoreType.DMA,
            pltpu.SemaphoreType.DMA,
        ],
        compiler_params=pltpu.CompilerParams(collective_id=0),
    )(x)


def _conv_silu(x, halo, k):
    b, s_per, c = x.shape
    n_taps = k.shape[0]

    def body(x_ref, halo_ref, k_ref, out_ref):
        ib = pl.program_id(0)
        kv = k_ref[...]
        k0 = kv[0, :][None, :]
        k1 = kv[1, :][None, :]
        k2 = kv[2, :][None, :]
        k3 = kv[3, :][None, :]

        ext = jnp.concatenate(
            [halo_ref[ib, :, :], x_ref[0, :, :]], axis=0
        )
        e1 = ext[1:, :]
        a = ext[:-1, :] * k0 + e1 * k1
        bv = ext[:-1, :] * k2 + e1 * k3
        acc = a[0:s_per, :] + bv[2:2 + s_per, :]
        out_ref[0, :, :] = acc * jax.nn.sigmoid(acc)

    return pl.pallas_call(
        body,
        grid=(b,),
        out_shape=jax.ShapeDtypeStruct((b, s_per, c), jnp.float32),
        in_specs=[
            pl.BlockSpec((1, s_per, c), lambda ib: (ib, 0, 0)),
            pl.BlockSpec((b, HALO, c), lambda ib: (0, 0, 0)),
            pl.BlockSpec((n_taps, c), lambda ib: (0, 0)),
        ],
        out_specs=pl.BlockSpec((1, s_per, c), lambda ib: (ib, 0, 0)),
    )(x, halo, k)


def kernel(x, k):
    halo = _halo_exchange(x)
    return _conv_silu(x, halo, k)
